# baseline (speedup 1.0000x reference)
import jax
import jax.numpy as jnp
from jax.experimental import pallas as pl
from jax.experimental.pallas import tpu as pltpu

EPS = 1e-5
ROWS = 2048


def _ln_kernel(x_ref, pos_ref, gamma_ref, beta_ref, out_ref):
    e = x_ref[0] + pos_ref[...]
    mean = jnp.mean(e, axis=-1, keepdims=True)
    c = e - mean
    var = jnp.mean(c * c, axis=-1, keepdims=True)
    inv = jax.lax.rsqrt(var + EPS)
    out_ref[0] = c * inv * gamma_ref[...] + beta_ref[...]


def kernel(x, pos_emb, ln_gamma, ln_beta):
    B, S, D = x.shape
    gamma2 = ln_gamma.reshape(1, D)
    beta2 = ln_beta.reshape(1, D)
    grid = (S // ROWS, B)
    return pl.pallas_call(
        _ln_kernel,
        grid=grid,
        in_specs=[
            pl.BlockSpec((1, ROWS, D), lambda j, b: (b, j, 0)),
            pl.BlockSpec((ROWS, D), lambda j, b: (j, 0)),
            pl.BlockSpec((1, D), lambda j, b: (0, 0)),
            pl.BlockSpec((1, D), lambda j, b: (0, 0)),
        ],
        out_specs=pl.BlockSpec((1, ROWS, D), lambda j, b: (b, j, 0)),
        out_shape=jax.ShapeDtypeStruct((B, S, D), x.dtype),
        compiler_params=pltpu.CompilerParams(
            dimension_semantics=("arbitrary", "arbitrary"),
        ),
    )(x, pos_emb[:S], gamma2, beta2)
